# baseline (speedup 1.0000x reference)
import jax
import jax.numpy as jnp
from jax import lax
from jax.experimental import pallas as pl
from jax.experimental.pallas import tpu as pltpu
from jax.experimental.pallas import tpu_sc as plsc

FIG = 640.0
IOU_T = 0.1
Q = 300
CHUNKS = 19
NC, NS = 2, 16
IMGS_PER_W = 2


def _body(logits_hbm, boxes_hbm, gt_hbm, probs_out, loss_out, stage_out,
          lbuf, bbuf, gbuf, obuf, redbuf, sem_l, sem_b, sem_g):
    c = lax.axis_index("c")
    s = lax.axis_index("s")
    wid = c * NS + s

    cp_l = pltpu.async_copy(
        logits_hbm.at[pl.ds(wid * 1200, 1200)], lbuf.at[pl.ds(0, 1200)], sem_l)
    cp_b = pltpu.async_copy(
        boxes_hbm.at[pl.ds(wid * 2400, 2400)], bbuf.at[pl.ds(0, 2400)], sem_b)
    cp_g = pltpu.async_copy(gt_hbm.at[pl.ds(wid * 8, 16)], gbuf, sem_g)
    cp_g.wait()
    cp_l.wait()
    cp_b.wait()

    lane = jnp.arange(16, dtype=jnp.int32)
    fzero = jnp.zeros((16,), jnp.float32)

    loss_vec = fzero
    num_row = fzero
    den_row = fzero
    gv = gbuf[...]
    for img in range(IMGS_PER_W):
        gx1 = jnp.sum(jnp.where(lane == 4 * img + 0, gv, 0.0))
        gy1 = jnp.sum(jnp.where(lane == 4 * img + 1, gv, 0.0))
        gx2 = jnp.sum(jnp.where(lane == 4 * img + 2, gv, 0.0))
        gy2 = jnp.sum(jnp.where(lane == 4 * img + 3, gv, 0.0))
        area2 = (gx2 - gx1) * (gy2 - gy1)

        l_acc, c_acc, s_acc = fzero, fzero, fzero
        for ci in range(CHUNKS):
            lidx = img * 600 + 2 * 16 * ci + 2 * lane
            l0 = plsc.load_gather(lbuf, [lidx])
            l1 = plsc.load_gather(lbuf, [lidx + 1])
            bidx = img * 1200 + 4 * 16 * ci + 4 * lane
            cx = plsc.load_gather(bbuf, [bidx])
            cy = plsc.load_gather(bbuf, [bidx + 1])
            w = plsc.load_gather(bbuf, [bidx + 2])
            h = plsc.load_gather(bbuf, [bidx + 3])

            d = l1 - l0
            pos = d > 0.0
            t = jnp.exp(-jnp.abs(d))
            x1 = (cx - 0.5 * w) * FIG
            y1 = (cy - 0.5 * h) * FIG
            x2 = (cx + 0.5 * w) * FIG
            y2 = (cy + 0.5 * h) * FIG
            area1 = (x2 - x1) * (y2 - y1)
            iw = jnp.maximum(jnp.minimum(x2, gx2) - jnp.maximum(x1, gx1), 0.0)
            ih = jnp.maximum(jnp.minimum(y2, gy2) - jnp.maximum(y1, gy1), 0.0)
            inter = iw * ih
            union = area1 + area2 - inter

            a = 1.0 + t
            b = t + 2.0
            ab = a * b
            r = 1.0 / (ab * union)
            ur = union * r
            prob = jnp.where(pos, b * ur, t * (b * ur))
            z = t * (a * ur)
            iou = inter * (ab * r)
            z2 = z * z
            poly = 1.0 + z2 * (1.0 / 3.0 + z2 * (0.2 + z2 * (1.0 / 7.0 + z2 * (1.0 / 9.0))))
            spl = jnp.maximum(d, 0.0) + (2.0 * z) * poly

            m = jnp.logical_and(iou >= IOU_T, pos)
            if ci == CHUNKS - 1:
                m = jnp.logical_and(m, lane < (Q - 16 * (CHUNKS - 1)))
            l_acc = l_acc + jnp.where(m, spl * iou, 0.0)
            c_acc = c_acc + jnp.where(m, 1.0, 0.0)
            s_acc = s_acc + jnp.where(m, prob, 0.0)

        loss_vec = loss_vec + l_acc
        num_row = jnp.where(lane == img, jnp.sum(s_acc), num_row)
        den_row = jnp.where(lane == img, jnp.sum(c_acc), den_row)

    obuf[...] = num_row / jnp.maximum(den_row, 1.0)
    pltpu.sync_copy(obuf, probs_out.at[wid])

    obuf[...] = loss_vec
    pltpu.sync_copy(obuf, stage_out.at[wid])
    plsc.subcore_barrier()

    @pl.when(s == 0)
    def _():
        pltpu.sync_copy(stage_out.at[pl.ds(c * NS, NS)], redbuf)
        acc = redbuf[0, :]
        for r_i in range(1, NS):
            acc = acc + redbuf[r_i, :]
        part = jnp.sum(acc * (1.0 / 64.0))
        obuf[...] = jnp.where(lane == 0, part, 0.0)
        pltpu.sync_copy(obuf, loss_out.at[c])


_sc_call = pl.kernel(
    _body,
    out_type=(
        jax.ShapeDtypeStruct((NC * NS, 16), jnp.float32),
        jax.ShapeDtypeStruct((NC, 16), jnp.float32),
        jax.ShapeDtypeStruct((NC * NS, 16), jnp.float32),
    ),
    mesh=plsc.VectorSubcoreMesh(
        core_axis_name="c", subcore_axis_name="s",
        num_cores=NC, num_subcores=NS),
    compiler_params=pltpu.CompilerParams(needs_layout_passes=False),
    scratch_types=[
        pltpu.VMEM((1216,), jnp.float32),
        pltpu.VMEM((2432,), jnp.float32),
        pltpu.VMEM((16,), jnp.float32),
        pltpu.VMEM((16,), jnp.float32),
        pltpu.VMEM((NS, 16), jnp.float32),
        pltpu.SemaphoreType.DMA,
        pltpu.SemaphoreType.DMA,
        pltpu.SemaphoreType.DMA,
    ],
)


@jax.jit
def kernel(logits, pred_boxes, gt):
    lf = logits.reshape(-1)
    bf = pred_boxes.reshape(-1)
    gf = jnp.pad(gt.reshape(-1), (0, 256))
    probs_rows, loss_part, _ = _sc_call(lf, bf, gf)
    det_loss = loss_part[0, 0] + loss_part[1, 0]
    max_probs = probs_rows[:, :2].reshape(64)
    return det_loss, max_probs


# speedup vs baseline: 1.0255x; 1.0255x over previous
import jax
import jax.numpy as jnp
from jax import lax
from jax.experimental import pallas as pl
from jax.experimental.pallas import tpu as pltpu
from jax.experimental.pallas import tpu_sc as plsc

FIG = 640.0
IOU_T = 0.1
Q = 300
CHUNKS = 19
NC, NS = 2, 16
IMGS_PER_W = 2
LOFF = 0
BOFF = 64 * 300 * 2
GOFF = BOFF + 64 * 300 * 4


def _body(flat_hbm, out, lbuf, bbuf, gbuf, obuf, redbuf, sem_l, sem_b, sem_g):
    c = lax.axis_index("c")
    s = lax.axis_index("s")
    wid = c * NS + s

    cp_l = pltpu.async_copy(
        flat_hbm.at[pl.ds(LOFF + wid * 1200, 1200)], lbuf.at[pl.ds(0, 1200)], sem_l)
    cp_b = pltpu.async_copy(
        flat_hbm.at[pl.ds(BOFF + wid * 2400, 2400)], bbuf.at[pl.ds(0, 2400)], sem_b)
    cp_g = pltpu.async_copy(flat_hbm.at[pl.ds(GOFF + wid * 8, 16)], gbuf, sem_g)
    cp_g.wait()
    cp_l.wait()
    cp_b.wait()

    lane = jnp.arange(16, dtype=jnp.int32)
    fzero = jnp.zeros((16,), jnp.float32)

    loss_vec = fzero
    num_row = fzero
    den_row = fzero
    gv = gbuf[...]
    for img in range(IMGS_PER_W):
        gx1 = jnp.sum(jnp.where(lane == 4 * img + 0, gv, 0.0))
        gy1 = jnp.sum(jnp.where(lane == 4 * img + 1, gv, 0.0))
        gx2 = jnp.sum(jnp.where(lane == 4 * img + 2, gv, 0.0))
        gy2 = jnp.sum(jnp.where(lane == 4 * img + 3, gv, 0.0))
        area2 = (gx2 - gx1) * (gy2 - gy1)

        l_acc, c_acc, s_acc = fzero, fzero, fzero
        for ci in range(CHUNKS):
            lidx = img * 600 + 2 * 16 * ci + 2 * lane
            l0 = plsc.load_gather(lbuf, [lidx])
            l1 = plsc.load_gather(lbuf, [lidx + 1])
            bidx = img * 1200 + 4 * 16 * ci + 4 * lane
            cx = plsc.load_gather(bbuf, [bidx])
            cy = plsc.load_gather(bbuf, [bidx + 1])
            w = plsc.load_gather(bbuf, [bidx + 2])
            h = plsc.load_gather(bbuf, [bidx + 3])

            d = l1 - l0
            pos = d > 0.0
            t = jnp.exp(-jnp.abs(d))
            x1 = (cx - 0.5 * w) * FIG
            y1 = (cy - 0.5 * h) * FIG
            x2 = (cx + 0.5 * w) * FIG
            y2 = (cy + 0.5 * h) * FIG
            area1 = (x2 - x1) * (y2 - y1)
            iw = jnp.maximum(jnp.minimum(x2, gx2) - jnp.maximum(x1, gx1), 0.0)
            ih = jnp.maximum(jnp.minimum(y2, gy2) - jnp.maximum(y1, gy1), 0.0)
            inter = iw * ih
            union = area1 + area2 - inter

            a = 1.0 + t
            b = t + 2.0
            ab = a * b
            r = 1.0 / (ab * union)
            ur = union * r
            prob = jnp.where(pos, b * ur, t * (b * ur))
            z = t * (a * ur)
            iou = inter * (ab * r)
            z2 = z * z
            poly = 1.0 + z2 * (1.0 / 3.0 + z2 * (0.2 + z2 * (1.0 / 7.0 + z2 * (1.0 / 9.0))))
            spl = jnp.maximum(d, 0.0) + (2.0 * z) * poly

            m = jnp.logical_and(iou >= IOU_T, pos)
            if ci == CHUNKS - 1:
                m = jnp.logical_and(m, lane < (Q - 16 * (CHUNKS - 1)))
            l_acc = l_acc + jnp.where(m, spl * iou, 0.0)
            c_acc = c_acc + jnp.where(m, 1.0, 0.0)
            s_acc = s_acc + jnp.where(m, prob, 0.0)

        loss_vec = loss_vec + l_acc
        num_row = jnp.where(lane == img, jnp.sum(s_acc), num_row)
        den_row = jnp.where(lane == img, jnp.sum(c_acc), den_row)

    mp_row = num_row / jnp.maximum(den_row, 1.0)
    row = jnp.where(lane == 2, jnp.sum(loss_vec), mp_row)
    obuf[...] = row
    pltpu.sync_copy(obuf, out.at[wid])
    plsc.subcore_barrier()

    @pl.when(s == 0)
    def _():
        pltpu.sync_copy(out.at[pl.ds(c * NS, NS)], redbuf)
        acc = redbuf[0, :]
        for r_i in range(1, NS):
            acc = acc + redbuf[r_i, :]
        part = jnp.sum(jnp.where(lane == 2, acc, 0.0)) * (1.0 / 64.0)
        obuf[...] = jnp.where(lane == 2, part, row)
        pltpu.sync_copy(obuf, out.at[wid])


_sc_call = pl.kernel(
    _body,
    out_type=jax.ShapeDtypeStruct((NC * NS, 16), jnp.float32),
    mesh=plsc.VectorSubcoreMesh(
        core_axis_name="c", subcore_axis_name="s",
        num_cores=NC, num_subcores=NS),
    compiler_params=pltpu.CompilerParams(needs_layout_passes=False),
    scratch_types=[
        pltpu.VMEM((1216,), jnp.float32),
        pltpu.VMEM((2432,), jnp.float32),
        pltpu.VMEM((16,), jnp.float32),
        pltpu.VMEM((16,), jnp.float32),
        pltpu.VMEM((NS, 16), jnp.float32),
        pltpu.SemaphoreType.DMA,
        pltpu.SemaphoreType.DMA,
        pltpu.SemaphoreType.DMA,
    ],
)


@jax.jit
def kernel(logits, pred_boxes, gt):
    flat = jnp.concatenate([
        logits.reshape(-1),
        pred_boxes.reshape(-1),
        gt.reshape(-1),
        jnp.zeros((256,), jnp.float32),
    ])
    out = _sc_call(flat)
    det_loss = out[0, 2] + out[NS, 2]
    max_probs = out[:, :2].reshape(64)
    return det_loss, max_probs
